# baseline (speedup 1.0000x reference)
import jax
import jax.numpy as jnp
from jax.experimental import pallas as pl
from jax.experimental.pallas import tpu as pltpu

N = 4194304
F = 16
COLS = 1024
ROWS = N // COLS
BROW = 512


def _tc_body(gs0_ref, a1_ref, a_ref, rh_ref, fgs_ref, out_ref):
    idx = fgs_ref[...]
    tbl_g = jnp.broadcast_to(gs0_ref[...], (BROW, F))
    tbl_a = jnp.broadcast_to(a1_ref[...], (BROW, F))
    g = jnp.take_along_axis(tbl_g, idx, axis=1, mode="promise_in_bounds")
    a = jnp.take_along_axis(tbl_a, idx, axis=1, mode="promise_in_bounds")
    out_ref[...] = g + a * a_ref[...] * rh_ref[...] * (1.0 / 420.0)


def kernel(gs0, a1, A, rh, FGs):
    A2 = A.reshape(ROWS, COLS)
    rh2 = rh.reshape(ROWS, COLS)
    fgs2 = FGs.reshape(ROWS, COLS)
    out = pl.pallas_call(
        _tc_body,
        out_shape=jax.ShapeDtypeStruct((ROWS, COLS), jnp.float32),
        grid=(ROWS // BROW,),
        in_specs=[
            pl.BlockSpec((F,), lambda i: (0,)),
            pl.BlockSpec((F,), lambda i: (0,)),
            pl.BlockSpec((BROW, COLS), lambda i: (i, 0)),
            pl.BlockSpec((BROW, COLS), lambda i: (i, 0)),
            pl.BlockSpec((BROW, COLS), lambda i: (i, 0)),
        ],
        out_specs=pl.BlockSpec((BROW, COLS), lambda i: (i, 0)),
    )(gs0, a1, A2, rh2, fgs2)
    return out.reshape(N)
